# baseline (speedup 1.0000x reference)
import functools

import jax
import jax.numpy as jnp
from jax.experimental import pallas as pl
from jax.experimental.pallas import tpu as pltpu

_NEG_BIG = -1e30


def _loss_kernel(d_ref, a_ref, l_ref, pv_ref, o_ref, md_ref, cnt_ref, acc_ref,
                 *, num_steps):
    i = pl.program_id(0)
    p = d_ref.shape[1]

    @pl.when(i == 0)
    def _init():
        cnt_ref[...] = jnp.zeros_like(cnt_ref)
        acc_ref[0] = _NEG_BIG
        acc_ref[1] = 0.0
        acc_ref[2] = 0.0
        acc_ref[4] = 0.0

    nb = d_ref.shape[0]
    d = d_ref[...]
    a = a_ref[...]
    lab = l_ref[...]
    mask = lab == 1

    hh = d.shape[2] // 2
    pmins, idxs = [], []
    for bi in range(nb):
        for hi in range(2):
            hs = slice(hi * hh, (hi + 1) * hh)
            pm = d[bi, 0, hs]
            fi = jnp.zeros(pm.shape, jnp.int32)
            for q in range(1, p):
                dq = d[bi, q, hs]
                better = dq < pm
                pm = jnp.where(better, dq, pm)
                fi = jnp.where(better, q, fi)
            pmins.append(pm)
            idxs.append(fi)
    pmin = jnp.concatenate(
        [jnp.concatenate(pmins[2 * bi:2 * bi + 2], axis=0)[None]
         for bi in range(nb)], axis=0)
    first_idx = jnp.concatenate(
        [jnp.concatenate(idxs[2 * bi:2 * bi + 2], axis=0)[None]
         for bi in range(nb)], axis=0)

    amax = jnp.max(a, axis=1)
    md_step = jnp.where(mask, pmin, -1.0)
    ma_step = jnp.where(mask, amax, _NEG_BIG)

    idx_m = jnp.where(mask, first_idx, p)
    iota_p = jax.lax.broadcasted_iota(jnp.int32, d.shape, 1)
    contrib = jnp.where(iota_p == idx_m[:, None], 1.0, 0.0)
    cnt_ref[...] += jnp.sum(contrib, axis=(0, 2))

    md_ref[pl.ds(i * nb, nb)] = md_step

    m_old = acc_ref[0]
    m_new = jnp.maximum(m_old, jnp.max(ma_step))
    scale = jnp.exp(m_old - m_new)
    e_step = jnp.exp(ma_step - m_new)
    acc_ref[0] = m_new
    acc_ref[1] = acc_ref[1] * scale + jnp.sum(jnp.sum(e_step, axis=0))
    acc_ref[2] = acc_ref[2] * scale + jnp.sum(jnp.sum(md_step * e_step, axis=0))
    acc_ref[4] += jnp.sum(jnp.sum(jnp.maximum(md_step, 0.0), axis=0))

    @pl.when(i == num_steps - 1)
    def _finalize():
        md = md_ref[...]
        counts = cnt_ref[...]

        n_f = jnp.sum(counts)
        n = n_f.astype(jnp.int32)
        safe_n = jnp.maximum(n_f, 1.0)
        base = acc_ref[4] / safe_n

        k = jnp.maximum(1, (3 * n) // 10)
        k_f = k.astype(jnp.float32)

        def search_bit(j, prefix):
            cand = prefix | (jnp.int32(1) << (30 - j))
            th = jax.lax.bitcast_convert_type(cand, jnp.float32)
            ind = jnp.where(md >= th, 1.0, 0.0)
            cnt = jnp.sum(jnp.sum(ind, axis=0))
            return jnp.where(cnt >= k_f, cand, prefix)

        prefix = jax.lax.fori_loop(0, 21, search_bit, jnp.int32(0))
        t = jax.lax.bitcast_convert_type(prefix, jnp.float32)
        hard_sum = jnp.sum(jnp.sum(jnp.maximum(md - t, 0.0), axis=0)) + k_f * t
        hard_loss = (hard_sum / k_f) * 2.0

        usage = jnp.sum(counts, axis=1) / n_f
        entropy = -jnp.sum(usage * jnp.log(usage + 1e-8))
        max_entropy = jnp.log(jnp.float32(p))
        usage_div = (max_entropy - entropy) * 0.1

        wcl = acc_ref[2] / acc_ref[1]

        cluster = base + hard_loss + usage_div + 0.5 * wcl
        cluster = jnp.where(n > 0, cluster, 0.0)

        pv = pv_ref[...]
        norms = jnp.maximum(
            jnp.sqrt(jnp.sum(pv * pv, axis=1, keepdims=True)), 1e-12)
        nv = pv / norms
        sim = jnp.dot(nv, nv.T, preferred_element_type=jnp.float32)
        rows = jax.lax.broadcasted_iota(jnp.int32, sim.shape, 0)
        cols = jax.lax.broadcasted_iota(jnp.int32, sim.shape, 1)
        offdiag = jnp.where(rows == cols, 0.0, jnp.abs(sim))
        div_loss = jnp.sum(offdiag) / jnp.float32(p * p)

        total = cluster + 0.01 * div_loss
        o_ref[...] = jnp.full((1, 1), total, dtype=jnp.float32)


def kernel(distances, activations, labels, prototype_vectors):
    b, p, h, w = distances.shape
    d = prototype_vectors.shape[1]
    labels32 = labels.astype(jnp.int32)
    nb = 4
    out = pl.pallas_call(
        functools.partial(_loss_kernel, num_steps=b // nb),
        grid=(b // nb,),
        in_specs=[
            pl.BlockSpec((nb, p, h, w), lambda i: (i, 0, 0, 0)),
            pl.BlockSpec((nb, p, h, w), lambda i: (i, 0, 0, 0)),
            pl.BlockSpec((nb, h, w), lambda i: (i, 0, 0)),
            pl.BlockSpec((p, d), lambda i: (0, 0)),
        ],
        out_specs=pl.BlockSpec((1, 1), lambda i: (0, 0)),
        out_shape=jax.ShapeDtypeStruct((1, 1), jnp.float32),
        scratch_shapes=[
            pltpu.VMEM((b, h, w), jnp.float32),
            pltpu.VMEM((p, w), jnp.float32),
            pltpu.SMEM((8,), jnp.float32),
        ],
    )(distances, activations, labels32, prototype_vectors)
    return out[0, 0]


# speedup vs baseline: 1.0766x; 1.0766x over previous
import functools

import jax
import jax.numpy as jnp
from jax.experimental import pallas as pl
from jax.experimental.pallas import tpu as pltpu

_NEG_BIG = -1e30


def _loss_kernel(d_ref, a_ref, l_ref, pv_ref, o_ref, md_ref, cnt_ref, acc_ref,
                 *, num_steps):
    i = pl.program_id(0)
    p = d_ref.shape[1]

    @pl.when(i == 0)
    def _init():
        cnt_ref[...] = jnp.zeros_like(cnt_ref)
        acc_ref[0] = _NEG_BIG
        acc_ref[1] = 0.0
        acc_ref[2] = 0.0
        acc_ref[4] = 0.0

    nb = d_ref.shape[0]
    d = d_ref[...]
    a = a_ref[...]
    lab = l_ref[...]
    mask = lab == 1

    hh = d.shape[2] // 2
    pmins, idxs = [], []
    for bi in range(nb):
        for hi in range(2):
            hs = slice(hi * hh, (hi + 1) * hh)
            pm = d[bi, 0, hs]
            fi = jnp.zeros(pm.shape, jnp.int32)
            for q in range(1, p):
                dq = d[bi, q, hs]
                better = dq < pm
                pm = jnp.where(better, dq, pm)
                fi = jnp.where(better, q, fi)
            pmins.append(pm)
            idxs.append(fi)
    pmin = jnp.concatenate(
        [jnp.concatenate(pmins[2 * bi:2 * bi + 2], axis=0)[None]
         for bi in range(nb)], axis=0)
    first_idx = jnp.concatenate(
        [jnp.concatenate(idxs[2 * bi:2 * bi + 2], axis=0)[None]
         for bi in range(nb)], axis=0)

    amax = jnp.max(a, axis=1)
    md_step = jnp.where(mask, pmin, -1.0)
    ma_step = jnp.where(mask, amax, _NEG_BIG)

    idx_m = jnp.where(mask, first_idx, p)
    iota_p = jax.lax.broadcasted_iota(jnp.int32, d.shape, 1)
    contrib = jnp.where(iota_p == idx_m[:, None], 1.0, 0.0)
    cnt_ref[...] += jnp.sum(contrib, axis=(0, 2))

    md_ref[pl.ds(i * nb, nb)] = md_step

    m_old = acc_ref[0]
    m_new = jnp.maximum(m_old, jnp.max(ma_step))
    scale = jnp.exp(m_old - m_new)
    e_step = jnp.exp(ma_step - m_new)
    acc_ref[0] = m_new
    acc_ref[1] = acc_ref[1] * scale + jnp.sum(jnp.sum(e_step, axis=0))
    acc_ref[2] = acc_ref[2] * scale + jnp.sum(jnp.sum(md_step * e_step, axis=0))
    acc_ref[4] += jnp.sum(jnp.sum(jnp.maximum(md_step, 0.0), axis=0))

    @pl.when(i == num_steps - 1)
    def _finalize():
        md = md_ref[...]
        counts = cnt_ref[...]

        n_f = jnp.sum(counts)
        n = n_f.astype(jnp.int32)
        safe_n = jnp.maximum(n_f, 1.0)
        base = acc_ref[4] / safe_n

        k = jnp.maximum(1, (3 * n) // 10)
        k_f = k.astype(jnp.float32)

        def search_bit(j, prefix):
            cand = prefix | (jnp.int32(1) << (30 - j))
            th = jax.lax.bitcast_convert_type(cand, jnp.float32)
            ind = jnp.where(md >= th, 1.0, 0.0)
            cnt = jnp.sum(jnp.sum(ind, axis=0))
            return jnp.where(cnt >= k_f, cand, prefix)

        prefix = jax.lax.fori_loop(0, 21, search_bit, jnp.int32(0))
        t = jax.lax.bitcast_convert_type(prefix, jnp.float32)
        hard_sum = jnp.sum(jnp.sum(jnp.maximum(md - t, 0.0), axis=0)) + k_f * t
        hard_loss = (hard_sum / k_f) * 2.0

        usage = jnp.sum(counts, axis=1) / n_f
        entropy = -jnp.sum(usage * jnp.log(usage + 1e-8))
        max_entropy = jnp.log(jnp.float32(p))
        usage_div = (max_entropy - entropy) * 0.1

        wcl = acc_ref[2] / acc_ref[1]

        cluster = base + hard_loss + usage_div + 0.5 * wcl
        cluster = jnp.where(n > 0, cluster, 0.0)

        pv = pv_ref[...]
        norms = jnp.maximum(
            jnp.sqrt(jnp.sum(pv * pv, axis=1, keepdims=True)), 1e-12)
        nv = pv / norms
        sim = jnp.dot(nv, nv.T, preferred_element_type=jnp.float32)
        rows = jax.lax.broadcasted_iota(jnp.int32, sim.shape, 0)
        cols = jax.lax.broadcasted_iota(jnp.int32, sim.shape, 1)
        offdiag = jnp.where(rows == cols, 0.0, jnp.abs(sim))
        div_loss = jnp.sum(offdiag) / jnp.float32(p * p)

        total = cluster + 0.01 * div_loss
        o_ref[...] = jnp.full((1, 1), total, dtype=jnp.float32)


def kernel(distances, activations, labels, prototype_vectors):
    b, p, h, w = distances.shape
    d = prototype_vectors.shape[1]
    labels32 = labels.astype(jnp.int32)
    nb = 2
    out = pl.pallas_call(
        functools.partial(_loss_kernel, num_steps=b // nb),
        grid=(b // nb,),
        in_specs=[
            pl.BlockSpec((nb, p, h, w), lambda i: (i, 0, 0, 0)),
            pl.BlockSpec((nb, p, h, w), lambda i: (i, 0, 0, 0)),
            pl.BlockSpec((nb, h, w), lambda i: (i, 0, 0)),
            pl.BlockSpec((p, d), lambda i: (0, 0)),
        ],
        out_specs=pl.BlockSpec((1, 1), lambda i: (0, 0)),
        out_shape=jax.ShapeDtypeStruct((1, 1), jnp.float32),
        scratch_shapes=[
            pltpu.VMEM((b, h, w), jnp.float32),
            pltpu.VMEM((p, w), jnp.float32),
            pltpu.SMEM((8,), jnp.float32),
        ],
    )(distances, activations, labels32, prototype_vectors)
    return out[0, 0]
